# baseline (speedup 1.0000x reference)
import functools

import jax
import jax.numpy as jnp
from jax import lax
from jax.experimental import pallas as pl
from jax.experimental.pallas import tpu as pltpu
from jax.experimental.pallas import tpu_sc as plsc

_B = 16384
_D = 64
_V = 1000000
_NC = 2
_NS = 16
_NW = _NC * _NS
_BPW = _B // _NW
_L = 128
_W = 11
_K = _W - 2
_NSLAB = 8
_SLAB = _BPW // _NSLAB
_G = _BPW // 16
_CHUNK = 128
_NCH = _BPW // _CHUNK


@functools.partial(
    pl.kernel,
    out_type=jax.ShapeDtypeStruct((_B, _D), jnp.float32),
    mesh=plsc.VectorSubcoreMesh(core_axis_name="c", subcore_axis_name="s"),
    scratch_types=[
        pltpu.VMEM((_BPW + 16,), jnp.int32),
        pltpu.VMEM((_W, _D, _L), jnp.float32),
        pltpu.VMEM((_SLAB, _D), jnp.float32),
        pltpu.SemaphoreType.DMA,
    ],
    compiler_params=pltpu.CompilerParams(
        use_tc_tiling_on_sc=True,
        needs_layout_passes=False,
        disable_bounds_checks=True,
    ),
)
def _sc_gather_sorted(idx_hbm, tab_hbm, out_hbm, idx_v, blk_v, rows_v, sem):
    wid = lax.axis_index("s") * _NC + lax.axis_index("c")
    base = wid * _BPW
    pltpu.sync_copy(idx_hbm.at[pl.ds(base, _BPW)], idx_v.at[pl.ds(0, _BPW)])
    pltpu.sync_copy(idx_hbm.at[pl.ds(base, 16)], idx_v.at[pl.ds(_BPW, 16)])

    def _c0_of(c):
        return pl.multiple_of(c * _L, _L)

    def _fetch(c, slot):
        pltpu.async_copy(
            tab_hbm.at[:, pl.ds(_c0_of(c), _L)], blk_v.at[slot], sem
        )

    def _drain_one():
        pltpu.make_async_copy(
            tab_hbm.at[:, pl.ds(0, _L)], blk_v.at[0], sem
        ).wait()

    dvecs = [lax.iota(jnp.int32, 16) + (16 * h) for h in range(_D // 16)]

    def _extract(l, slot, j):
        lvec = jnp.full((16,), l, jnp.int32)
        blk = blk_v.at[slot]
        row = rows_v.at[j]
        for h in range(_D // 16):
            row[pl.ds(16 * h, 16)] = plsc.load_gather(blk, [dvecs[h], lvec])

    neg1 = jnp.int32(-1)

    vec0 = idx_v[pl.ds(0, 16)]
    dc_f = jnp.int32(0)
    prev_c_f = neg1
    for t in range(_K):
        c_t = vec0[t] // _L
        is_new = c_t != prev_c_f

        @pl.when(is_new)
        def _(c_t=c_t, dc_f=dc_f):
            _fetch(c_t, dc_f % _W)

        dc_f = dc_f + is_new.astype(jnp.int32)
        prev_c_f = c_t

    def body(g, carry):
        dc_f, prev_c_f, dc_d, prev_c_d = carry
        cur = idx_v[pl.ds(g * 16, 16)]
        nxt = idx_v[pl.ds(g * 16 + 16, 16)]
        for t in range(16):
            r = cur[t]
            c_t = r // _L
            is_new_d = c_t != prev_c_d

            @pl.when(is_new_d)
            def _():
                _drain_one()

            dc_d = dc_d + is_new_d.astype(jnp.int32)
            prev_c_d = c_t
            _extract(r - c_t * _L, (dc_d - 1) % _W, (g % (_G // _NSLAB)) * 16 + t)
            r_a = cur[t + _K] if t + _K < 16 else nxt[t + _K - 16]
            c_a = r_a // _L
            is_new_f = c_a != prev_c_f

            @pl.when(is_new_f)
            def _(c_a=c_a, dc_f=dc_f):
                _fetch(c_a, dc_f % _W)

            dc_f = dc_f + is_new_f.astype(jnp.int32)
            prev_c_f = c_a

        @pl.when(g % (_G // _NSLAB) == _G // _NSLAB - 1)
        def _():
            slab = g // (_G // _NSLAB)
            pltpu.sync_copy(
                rows_v, out_hbm.at[pl.ds(base + slab * _SLAB, _SLAB), :]
            )

        return dc_f, prev_c_f, dc_d, prev_c_d

    dc_f, _, dc_d, _ = lax.fori_loop(
        0, _G, body, (dc_f, prev_c_f, jnp.int32(0), neg1)
    )

    lax.fori_loop(0, dc_f - dc_d, lambda i, c: (_drain_one(), c)[1], 0)


@functools.partial(
    pl.kernel,
    out_type=jax.ShapeDtypeStruct((_B, _D), jnp.float32),
    mesh=plsc.VectorSubcoreMesh(core_axis_name="c", subcore_axis_name="s"),
    scratch_types=[
        pltpu.VMEM((_NCH, _CHUNK), jnp.int32),
        pltpu.VMEM((_BPW, _D), jnp.float32),
        pltpu.SemaphoreType.DMA,
    ],
    compiler_params=pltpu.CompilerParams(use_tc_tiling_on_sc=False),
)
def _sc_scatter(pos_hbm, rows_hbm, out_hbm, pos_v, rows_v, sem):
    wid = lax.axis_index("s") * _NC + lax.axis_index("c")
    base = wid * _BPW
    for j in range(_NCH):
        pltpu.sync_copy(
            pos_hbm.at[pl.ds(base + j * _CHUNK, _CHUNK)], pos_v.at[j]
        )
    pltpu.sync_copy(rows_hbm.at[pl.ds(base, _BPW), :], rows_v)
    copies = []
    for j in range(_NCH):
        copies.append(
            pltpu.async_copy(
                rows_v.at[pl.ds(j * _CHUNK, _CHUNK), :],
                out_hbm.at[pos_v.at[j]],
                sem,
            )
        )
    for cp in copies:
        cp.wait()


def kernel(token_ids, embedding):
    ids = token_ids.astype(jnp.int32)
    packed = ((ids // _L) << 14) | lax.iota(jnp.int32, _B)
    packed_sorted = lax.sort(packed)
    order = packed_sorted & (_B - 1)
    ids_sorted = jnp.take(ids, order, axis=0)
    rows_sorted = _sc_gather_sorted(ids_sorted, embedding.T)
    return _sc_scatter(order, rows_sorted)


# speedup vs baseline: 1.0485x; 1.0485x over previous
import functools

import jax
import jax.numpy as jnp
from jax import lax
from jax.experimental import pallas as pl
from jax.experimental.pallas import tpu as pltpu
from jax.experimental.pallas import tpu_sc as plsc

_B = 16384
_D = 64
_V = 1000000
_NC = 2
_NS = 16
_NW = _NC * _NS
_BPW = _B // _NW
_L = 128
_W = 11
_K = _W - 2
_NSLAB = 8
_SLAB = _BPW // _NSLAB
_G = _BPW // 16
_CHUNK = 128
_NCH = _BPW // _CHUNK


@functools.partial(
    pl.kernel,
    out_type=jax.ShapeDtypeStruct((_B * _D,), jnp.float32),
    mesh=plsc.VectorSubcoreMesh(core_axis_name="c", subcore_axis_name="s"),
    scratch_types=[
        pltpu.VMEM((_BPW + 16,), jnp.int32),
        pltpu.VMEM((_W, _D, _L), jnp.float32),
        pltpu.VMEM((_SLAB * _D,), jnp.float32),
        pltpu.SemaphoreType.DMA,
    ],
    compiler_params=pltpu.CompilerParams(
        use_tc_tiling_on_sc=True,
        needs_layout_passes=False,
        disable_bounds_checks=True,
    ),
)
def _sc_gather_sorted(idx_hbm, tab_hbm, out_hbm, idx_v, blk_v, rows_v, sem):
    wid = lax.axis_index("s") * _NC + lax.axis_index("c")
    base = wid * _BPW
    pltpu.sync_copy(idx_hbm.at[pl.ds(base, _BPW)], idx_v.at[pl.ds(0, _BPW)])
    pltpu.sync_copy(idx_hbm.at[pl.ds(base, 16)], idx_v.at[pl.ds(_BPW, 16)])

    def _c0_of(c):
        return pl.multiple_of(c * _L, _L)

    def _fetch(c, slot):
        pltpu.async_copy(
            tab_hbm.at[:, pl.ds(_c0_of(c), _L)], blk_v.at[slot], sem
        )

    def _drain_one():
        pltpu.make_async_copy(
            tab_hbm.at[:, pl.ds(0, _L)], blk_v.at[0], sem
        ).wait()

    dvecs = [lax.iota(jnp.int32, 16) + (16 * h) for h in range(_D // 16)]

    def _extract(l, slot, j):
        lvec = jnp.full((16,), l, jnp.int32)
        blk = blk_v.at[slot]
        for h in range(_D // 16):
            rows_v[pl.ds(j * _D + 16 * h, 16)] = plsc.load_gather(
                blk, [dvecs[h], lvec]
            )

    neg1 = jnp.int32(-1)

    vec0 = idx_v[pl.ds(0, 16)]
    dc_f = jnp.int32(0)
    prev_c_f = neg1
    for t in range(_K):
        c_t = vec0[t] // _L
        is_new = c_t != prev_c_f

        @pl.when(is_new)
        def _(c_t=c_t, dc_f=dc_f):
            _fetch(c_t, dc_f % _W)

        dc_f = dc_f + is_new.astype(jnp.int32)
        prev_c_f = c_t

    def body(g, carry):
        dc_f, prev_c_f, dc_d, prev_c_d = carry
        cur = idx_v[pl.ds(g * 16, 16)]
        nxt = idx_v[pl.ds(g * 16 + 16, 16)]
        for t in range(16):
            r = cur[t]
            c_t = r // _L
            is_new_d = c_t != prev_c_d

            @pl.when(is_new_d)
            def _():
                _drain_one()

            dc_d = dc_d + is_new_d.astype(jnp.int32)
            prev_c_d = c_t
            _extract(r - c_t * _L, (dc_d - 1) % _W, (g % (_G // _NSLAB)) * 16 + t)
            r_a = cur[t + _K] if t + _K < 16 else nxt[t + _K - 16]
            c_a = r_a // _L
            is_new_f = c_a != prev_c_f

            @pl.when(is_new_f)
            def _(c_a=c_a, dc_f=dc_f):
                _fetch(c_a, dc_f % _W)

            dc_f = dc_f + is_new_f.astype(jnp.int32)
            prev_c_f = c_a

        @pl.when(g % (_G // _NSLAB) == _G // _NSLAB - 1)
        def _():
            slab = g // (_G // _NSLAB)
            pltpu.sync_copy(
                rows_v,
                out_hbm.at[pl.ds((base + slab * _SLAB) * _D, _SLAB * _D)],
            )

        return dc_f, prev_c_f, dc_d, prev_c_d

    dc_f, _, dc_d, _ = lax.fori_loop(
        0, _G, body, (dc_f, prev_c_f, jnp.int32(0), neg1)
    )

    lax.fori_loop(0, dc_f - dc_d, lambda i, c: (_drain_one(), c)[1], 0)


@functools.partial(
    pl.kernel,
    out_type=jax.ShapeDtypeStruct((_B, _D), jnp.float32),
    mesh=plsc.VectorSubcoreMesh(core_axis_name="c", subcore_axis_name="s"),
    scratch_types=[
        pltpu.VMEM((_NCH, _CHUNK), jnp.int32),
        pltpu.VMEM((_BPW, _D), jnp.float32),
        pltpu.SemaphoreType.DMA,
    ],
    compiler_params=pltpu.CompilerParams(use_tc_tiling_on_sc=False),
)
def _sc_scatter(pos_hbm, rows_hbm, out_hbm, pos_v, rows_v, sem):
    wid = lax.axis_index("s") * _NC + lax.axis_index("c")
    base = wid * _BPW
    for j in range(_NCH):
        pltpu.sync_copy(
            pos_hbm.at[pl.ds(base + j * _CHUNK, _CHUNK)], pos_v.at[j]
        )
    pltpu.sync_copy(rows_hbm.at[pl.ds(base, _BPW), :], rows_v)
    copies = []
    for j in range(_NCH):
        copies.append(
            pltpu.async_copy(
                rows_v.at[pl.ds(j * _CHUNK, _CHUNK), :],
                out_hbm.at[pos_v.at[j]],
                sem,
            )
        )
    for cp in copies:
        cp.wait()


def kernel(token_ids, embedding):
    ids = token_ids.astype(jnp.int32)
    packed = ((ids // _L) << 14) | lax.iota(jnp.int32, _B)
    packed_sorted = lax.sort(packed)
    order = packed_sorted & (_B - 1)
    ids_sorted = jnp.take(ids, order, axis=0)
    rows_sorted = _sc_gather_sorted(ids_sorted, embedding.T)
    return _sc_scatter(order, rows_sorted.reshape(_B, _D))


# speedup vs baseline: 1.0726x; 1.0229x over previous
import functools

import jax
import jax.numpy as jnp
from jax import lax
from jax.experimental import pallas as pl
from jax.experimental.pallas import tpu as pltpu
from jax.experimental.pallas import tpu_sc as plsc

_B = 16384
_D = 64
_V = 1000000
_NC = 2
_NS = 16
_NW = _NC * _NS
_BPW = _B // _NW
_L = 128
_W = 12
_K = _W - 2
_NSLAB = 8
_SLAB = _BPW // _NSLAB
_G = _BPW // 16
_CHUNK = 128
_NCH = _BPW // _CHUNK


@functools.partial(
    pl.kernel,
    out_type=jax.ShapeDtypeStruct((_B * _D,), jnp.float32),
    mesh=plsc.VectorSubcoreMesh(core_axis_name="c", subcore_axis_name="s"),
    scratch_types=[
        pltpu.VMEM((_BPW + 16,), jnp.int32),
        pltpu.VMEM((_W, _D, _L), jnp.float32),
        pltpu.VMEM((_SLAB * _D,), jnp.float32),
        pltpu.SemaphoreType.DMA,
    ],
    compiler_params=pltpu.CompilerParams(
        use_tc_tiling_on_sc=True,
        needs_layout_passes=False,
        disable_bounds_checks=True,
    ),
)
def _sc_gather_sorted(idx_hbm, tab_hbm, out_hbm, idx_v, blk_v, rows_v, sem):
    wid = lax.axis_index("s") * _NC + lax.axis_index("c")
    base = wid * _BPW
    pltpu.sync_copy(idx_hbm.at[pl.ds(base, _BPW)], idx_v.at[pl.ds(0, _BPW)])
    pltpu.sync_copy(idx_hbm.at[pl.ds(base, 16)], idx_v.at[pl.ds(_BPW, 16)])

    def _c0_of(c):
        return pl.multiple_of(c * _L, _L)

    def _fetch(c, slot):
        pltpu.async_copy(
            tab_hbm.at[:, pl.ds(_c0_of(c), _L)], blk_v.at[slot], sem
        )

    def _drain_one():
        pltpu.make_async_copy(
            tab_hbm.at[:, pl.ds(0, _L)], blk_v.at[0], sem
        ).wait()

    dvecs = [lax.iota(jnp.int32, 16) + (16 * h) for h in range(_D // 16)]

    def _extract(l, slot, j):
        lvec = jnp.full((16,), l, jnp.int32)
        blk = blk_v.at[slot]
        for h in range(_D // 16):
            rows_v[pl.ds(j * _D + 16 * h, 16)] = plsc.load_gather(
                blk, [dvecs[h], lvec]
            )

    neg1 = jnp.int32(-1)

    vec0 = idx_v[pl.ds(0, 16)]
    dc_f = jnp.int32(0)
    prev_c_f = neg1
    for t in range(_K):
        c_t = vec0[t] // _L
        is_new = c_t != prev_c_f

        @pl.when(is_new)
        def _(c_t=c_t, dc_f=dc_f):
            _fetch(c_t, dc_f % _W)

        dc_f = dc_f + is_new.astype(jnp.int32)
        prev_c_f = c_t

    def body(g, carry):
        dc_f, prev_c_f, dc_d, prev_c_d = carry
        cur = idx_v[pl.ds(g * 16, 16)]
        nxt = idx_v[pl.ds(g * 16 + 16, 16)]
        for t in range(16):
            r = cur[t]
            c_t = r // _L
            is_new_d = c_t != prev_c_d

            @pl.when(is_new_d)
            def _():
                _drain_one()

            dc_d = dc_d + is_new_d.astype(jnp.int32)
            prev_c_d = c_t
            _extract(r - c_t * _L, (dc_d - 1) % _W, (g % (_G // _NSLAB)) * 16 + t)
            r_a = cur[t + _K] if t + _K < 16 else nxt[t + _K - 16]
            c_a = r_a // _L
            is_new_f = c_a != prev_c_f

            @pl.when(is_new_f)
            def _(c_a=c_a, dc_f=dc_f):
                _fetch(c_a, dc_f % _W)

            dc_f = dc_f + is_new_f.astype(jnp.int32)
            prev_c_f = c_a

        @pl.when(g % (_G // _NSLAB) == _G // _NSLAB - 1)
        def _():
            slab = g // (_G // _NSLAB)
            pltpu.sync_copy(
                rows_v,
                out_hbm.at[pl.ds((base + slab * _SLAB) * _D, _SLAB * _D)],
            )

        return dc_f, prev_c_f, dc_d, prev_c_d

    dc_f, _, dc_d, _ = lax.fori_loop(
        0, _G, body, (dc_f, prev_c_f, jnp.int32(0), neg1)
    )

    lax.fori_loop(0, dc_f - dc_d, lambda i, c: (_drain_one(), c)[1], 0)


@functools.partial(
    pl.kernel,
    out_type=jax.ShapeDtypeStruct((_B, _D), jnp.float32),
    mesh=plsc.VectorSubcoreMesh(core_axis_name="c", subcore_axis_name="s"),
    scratch_types=[
        pltpu.VMEM((_NCH, _CHUNK), jnp.int32),
        pltpu.VMEM((_BPW, _D), jnp.float32),
        pltpu.SemaphoreType.DMA,
    ],
    compiler_params=pltpu.CompilerParams(use_tc_tiling_on_sc=False),
)
def _sc_scatter(pos_hbm, rows_hbm, out_hbm, pos_v, rows_v, sem):
    wid = lax.axis_index("s") * _NC + lax.axis_index("c")
    base = wid * _BPW
    for j in range(_NCH):
        pltpu.sync_copy(
            pos_hbm.at[pl.ds(base + j * _CHUNK, _CHUNK)], pos_v.at[j]
        )
    pltpu.sync_copy(rows_hbm.at[pl.ds(base, _BPW), :], rows_v)
    copies = []
    for j in range(_NCH):
        copies.append(
            pltpu.async_copy(
                rows_v.at[pl.ds(j * _CHUNK, _CHUNK), :],
                out_hbm.at[pos_v.at[j]],
                sem,
            )
        )
    for cp in copies:
        cp.wait()


def kernel(token_ids, embedding):
    ids = token_ids.astype(jnp.int32)
    packed = ((ids // _L) << 14) | lax.iota(jnp.int32, _B)
    packed_sorted = lax.sort(packed)
    order = packed_sorted & (_B - 1)
    ids_sorted = jnp.take(ids, order, axis=0)
    rows_sorted = _sc_gather_sorted(ids_sorted, embedding.T)
    return _sc_scatter(order, rows_sorted.reshape(_B, _D))


# speedup vs baseline: 1.0851x; 1.0117x over previous
import functools

import jax
import jax.numpy as jnp
from jax import lax
from jax.experimental import pallas as pl
from jax.experimental.pallas import tpu as pltpu
from jax.experimental.pallas import tpu_sc as plsc

_B = 16384
_D = 64
_V = 1000000
_NC = 2
_NS = 16
_NW = _NC * _NS
_BPW = _B // _NW
_L = 128
_W = 13
_K = _W - 2
_NSLAB = 8
_SLAB = _BPW // _NSLAB
_G = _BPW // 16
_CHUNK = 128
_NCH = _BPW // _CHUNK


@functools.partial(
    pl.kernel,
    out_type=jax.ShapeDtypeStruct((_B * _D,), jnp.float32),
    mesh=plsc.VectorSubcoreMesh(core_axis_name="c", subcore_axis_name="s"),
    scratch_types=[
        pltpu.VMEM((_BPW + 16,), jnp.int32),
        pltpu.VMEM((_W, _D, _L), jnp.float32),
        pltpu.VMEM((_SLAB * _D,), jnp.float32),
        pltpu.SemaphoreType.DMA,
    ],
    compiler_params=pltpu.CompilerParams(
        use_tc_tiling_on_sc=True,
        needs_layout_passes=False,
        disable_bounds_checks=True,
    ),
)
def _sc_gather_sorted(idx_hbm, tab_hbm, out_hbm, idx_v, blk_v, rows_v, sem):
    wid = lax.axis_index("s") * _NC + lax.axis_index("c")
    base = wid * _BPW
    pltpu.sync_copy(idx_hbm.at[pl.ds(base, _BPW)], idx_v.at[pl.ds(0, _BPW)])
    pltpu.sync_copy(idx_hbm.at[pl.ds(base, 16)], idx_v.at[pl.ds(_BPW, 16)])

    def _c0_of(c):
        return pl.multiple_of(c * _L, _L)

    def _fetch(c, slot):
        pltpu.async_copy(
            tab_hbm.at[:, pl.ds(_c0_of(c), _L)], blk_v.at[slot], sem
        )

    def _drain_one():
        pltpu.make_async_copy(
            tab_hbm.at[:, pl.ds(0, _L)], blk_v.at[0], sem
        ).wait()

    dvecs = [lax.iota(jnp.int32, 16) + (16 * h) for h in range(_D // 16)]

    def _extract(l, slot, j):
        lvec = jnp.full((16,), l, jnp.int32)
        blk = blk_v.at[slot]
        for h in range(_D // 16):
            rows_v[pl.ds(j * _D + 16 * h, 16)] = plsc.load_gather(
                blk, [dvecs[h], lvec]
            )

    neg1 = jnp.int32(-1)

    vec0 = idx_v[pl.ds(0, 16)]
    dc_f = jnp.int32(0)
    prev_c_f = neg1
    for t in range(_K):
        c_t = vec0[t] // _L
        is_new = c_t != prev_c_f

        @pl.when(is_new)
        def _(c_t=c_t, dc_f=dc_f):
            _fetch(c_t, dc_f % _W)

        dc_f = dc_f + is_new.astype(jnp.int32)
        prev_c_f = c_t

    def body(g, carry):
        dc_f, prev_c_f, dc_d, prev_c_d = carry
        cur = idx_v[pl.ds(g * 16, 16)]
        nxt = idx_v[pl.ds(g * 16 + 16, 16)]
        for t in range(16):
            r = cur[t]
            c_t = r // _L
            is_new_d = c_t != prev_c_d

            @pl.when(is_new_d)
            def _():
                _drain_one()

            dc_d = dc_d + is_new_d.astype(jnp.int32)
            prev_c_d = c_t
            _extract(r - c_t * _L, (dc_d - 1) % _W, (g % (_G // _NSLAB)) * 16 + t)
            r_a = cur[t + _K] if t + _K < 16 else nxt[t + _K - 16]
            c_a = r_a // _L
            is_new_f = c_a != prev_c_f

            @pl.when(is_new_f)
            def _(c_a=c_a, dc_f=dc_f):
                _fetch(c_a, dc_f % _W)

            dc_f = dc_f + is_new_f.astype(jnp.int32)
            prev_c_f = c_a

        @pl.when(g % (_G // _NSLAB) == _G // _NSLAB - 1)
        def _():
            slab = g // (_G // _NSLAB)
            pltpu.sync_copy(
                rows_v,
                out_hbm.at[pl.ds((base + slab * _SLAB) * _D, _SLAB * _D)],
            )

        return dc_f, prev_c_f, dc_d, prev_c_d

    dc_f, _, dc_d, _ = lax.fori_loop(
        0, _G, body, (dc_f, prev_c_f, jnp.int32(0), neg1)
    )

    lax.fori_loop(0, dc_f - dc_d, lambda i, c: (_drain_one(), c)[1], 0)


@functools.partial(
    pl.kernel,
    out_type=jax.ShapeDtypeStruct((_B, _D), jnp.float32),
    mesh=plsc.VectorSubcoreMesh(core_axis_name="c", subcore_axis_name="s"),
    scratch_types=[
        pltpu.VMEM((_NCH, _CHUNK), jnp.int32),
        pltpu.VMEM((_BPW, _D), jnp.float32),
        pltpu.SemaphoreType.DMA,
    ],
    compiler_params=pltpu.CompilerParams(use_tc_tiling_on_sc=False),
)
def _sc_scatter(pos_hbm, rows_hbm, out_hbm, pos_v, rows_v, sem):
    wid = lax.axis_index("s") * _NC + lax.axis_index("c")
    base = wid * _BPW
    for j in range(_NCH):
        pltpu.sync_copy(
            pos_hbm.at[pl.ds(base + j * _CHUNK, _CHUNK)], pos_v.at[j]
        )
    pltpu.sync_copy(rows_hbm.at[pl.ds(base, _BPW), :], rows_v)
    copies = []
    for j in range(_NCH):
        copies.append(
            pltpu.async_copy(
                rows_v.at[pl.ds(j * _CHUNK, _CHUNK), :],
                out_hbm.at[pos_v.at[j]],
                sem,
            )
        )
    for cp in copies:
        cp.wait()


def kernel(token_ids, embedding):
    ids = token_ids.astype(jnp.int32)
    packed = ((ids // _L) << 14) | lax.iota(jnp.int32, _B)
    packed_sorted = lax.sort(packed)
    order = packed_sorted & (_B - 1)
    ids_sorted = jnp.take(ids, order, axis=0)
    rows_sorted = _sc_gather_sorted(ids_sorted, embedding.T)
    return _sc_scatter(order, rows_sorted.reshape(_B, _D))


# speedup vs baseline: 1.0929x; 1.0072x over previous
import functools

import jax
import jax.numpy as jnp
from jax import lax
from jax.experimental import pallas as pl
from jax.experimental.pallas import tpu as pltpu
from jax.experimental.pallas import tpu_sc as plsc

_B = 16384
_D = 64
_V = 1000000
_NC = 2
_NS = 16
_NW = _NC * _NS
_BPW = _B // _NW
_L = 128
_W = 14
_K = _W - 2
_NSLAB = 8
_SLAB = _BPW // _NSLAB
_G = _BPW // 16
_CHUNK = 128
_NCH = _BPW // _CHUNK


@functools.partial(
    pl.kernel,
    out_type=jax.ShapeDtypeStruct((_B * _D,), jnp.float32),
    mesh=plsc.VectorSubcoreMesh(core_axis_name="c", subcore_axis_name="s"),
    scratch_types=[
        pltpu.VMEM((_BPW + 16,), jnp.int32),
        pltpu.VMEM((_W, _D, _L), jnp.float32),
        pltpu.VMEM((_SLAB * _D,), jnp.float32),
        pltpu.SemaphoreType.DMA,
    ],
    compiler_params=pltpu.CompilerParams(
        use_tc_tiling_on_sc=True,
        needs_layout_passes=False,
        disable_bounds_checks=True,
    ),
)
def _sc_gather_sorted(idx_hbm, tab_hbm, out_hbm, idx_v, blk_v, rows_v, sem):
    wid = lax.axis_index("s") * _NC + lax.axis_index("c")
    base = wid * _BPW
    pltpu.sync_copy(idx_hbm.at[pl.ds(base, _BPW)], idx_v.at[pl.ds(0, _BPW)])
    pltpu.sync_copy(idx_hbm.at[pl.ds(base, 16)], idx_v.at[pl.ds(_BPW, 16)])

    def _c0_of(c):
        return pl.multiple_of(c * _L, _L)

    def _fetch(c, slot):
        pltpu.async_copy(
            tab_hbm.at[:, pl.ds(_c0_of(c), _L)], blk_v.at[slot], sem
        )

    def _drain_one():
        pltpu.make_async_copy(
            tab_hbm.at[:, pl.ds(0, _L)], blk_v.at[0], sem
        ).wait()

    dvecs = [lax.iota(jnp.int32, 16) + (16 * h) for h in range(_D // 16)]

    def _extract(l, slot, j):
        lvec = jnp.full((16,), l, jnp.int32)
        blk = blk_v.at[slot]
        for h in range(_D // 16):
            rows_v[pl.ds(j * _D + 16 * h, 16)] = plsc.load_gather(
                blk, [dvecs[h], lvec]
            )

    neg1 = jnp.int32(-1)

    vec0 = idx_v[pl.ds(0, 16)]
    dc_f = jnp.int32(0)
    prev_c_f = neg1
    for t in range(_K):
        c_t = vec0[t] // _L
        is_new = c_t != prev_c_f

        @pl.when(is_new)
        def _(c_t=c_t, dc_f=dc_f):
            _fetch(c_t, dc_f % _W)

        dc_f = dc_f + is_new.astype(jnp.int32)
        prev_c_f = c_t

    def body(g, carry):
        dc_f, prev_c_f, dc_d, prev_c_d = carry
        cur = idx_v[pl.ds(g * 16, 16)]
        nxt = idx_v[pl.ds(g * 16 + 16, 16)]
        for t in range(16):
            r = cur[t]
            c_t = r // _L
            is_new_d = c_t != prev_c_d

            @pl.when(is_new_d)
            def _():
                _drain_one()

            dc_d = dc_d + is_new_d.astype(jnp.int32)
            prev_c_d = c_t
            _extract(r - c_t * _L, (dc_d - 1) % _W, (g % (_G // _NSLAB)) * 16 + t)
            r_a = cur[t + _K] if t + _K < 16 else nxt[t + _K - 16]
            c_a = r_a // _L
            is_new_f = c_a != prev_c_f

            @pl.when(is_new_f)
            def _(c_a=c_a, dc_f=dc_f):
                _fetch(c_a, dc_f % _W)

            dc_f = dc_f + is_new_f.astype(jnp.int32)
            prev_c_f = c_a

        @pl.when(g % (_G // _NSLAB) == _G // _NSLAB - 1)
        def _():
            slab = g // (_G // _NSLAB)
            pltpu.sync_copy(
                rows_v,
                out_hbm.at[pl.ds((base + slab * _SLAB) * _D, _SLAB * _D)],
            )

        return dc_f, prev_c_f, dc_d, prev_c_d

    dc_f, _, dc_d, _ = lax.fori_loop(
        0, _G, body, (dc_f, prev_c_f, jnp.int32(0), neg1)
    )

    lax.fori_loop(0, dc_f - dc_d, lambda i, c: (_drain_one(), c)[1], 0)


@functools.partial(
    pl.kernel,
    out_type=jax.ShapeDtypeStruct((_B, _D), jnp.float32),
    mesh=plsc.VectorSubcoreMesh(core_axis_name="c", subcore_axis_name="s"),
    scratch_types=[
        pltpu.VMEM((_NCH, _CHUNK), jnp.int32),
        pltpu.VMEM((_BPW, _D), jnp.float32),
        pltpu.SemaphoreType.DMA,
    ],
    compiler_params=pltpu.CompilerParams(use_tc_tiling_on_sc=False),
)
def _sc_scatter(pos_hbm, rows_hbm, out_hbm, pos_v, rows_v, sem):
    wid = lax.axis_index("s") * _NC + lax.axis_index("c")
    base = wid * _BPW
    for j in range(_NCH):
        pltpu.sync_copy(
            pos_hbm.at[pl.ds(base + j * _CHUNK, _CHUNK)], pos_v.at[j]
        )
    pltpu.sync_copy(rows_hbm.at[pl.ds(base, _BPW), :], rows_v)
    copies = []
    for j in range(_NCH):
        copies.append(
            pltpu.async_copy(
                rows_v.at[pl.ds(j * _CHUNK, _CHUNK), :],
                out_hbm.at[pos_v.at[j]],
                sem,
            )
        )
    for cp in copies:
        cp.wait()


def kernel(token_ids, embedding):
    ids = token_ids.astype(jnp.int32)
    packed = ((ids // _L) << 14) | lax.iota(jnp.int32, _B)
    packed_sorted = lax.sort(packed)
    order = packed_sorted & (_B - 1)
    ids_sorted = jnp.take(ids, order, axis=0)
    rows_sorted = _sc_gather_sorted(ids_sorted, embedding.T)
    return _sc_scatter(order, rows_sorted.reshape(_B, _D))
